# speedup vs baseline: 20.3470x; 20.3470x over previous
import functools

import jax
import jax.numpy as jnp
from jax import lax
from jax.experimental import pallas as pl
from jax.experimental.pallas import tpu as pltpu
from jax.experimental.pallas import tpu_sc as plsc

N = 10000
NPAD = 10240
D = 128
E = 320000
C = 128
NCHUNK = E // C
NC = 2
NS = 16
NW = NC * NS
STRIPE = NPAD // NS

_f32 = jnp.float32


def _sc_degree(dst):
    mesh = plsc.VectorSubcoreMesh(core_axis_name="c", subcore_axis_name="s")

    @functools.partial(
        pl.kernel,
        out_type=jax.ShapeDtypeStruct((NC, NPAD), _f32),
        mesh=mesh,
        scratch_types=[
            pltpu.VMEM_SHARED((NPAD,), _f32),
            pltpu.VMEM((C,), jnp.int32),
            pltpu.VMEM((C,), _f32),
            pltpu.VMEM((STRIPE,), _f32),
        ],
    )
    def deg_kernel(dst_hbm, out_hbm, hist, idx_v, ones_v, zer_v):
        cid = lax.axis_index("c")
        sid = lax.axis_index("s")
        wid = cid * NS + sid

        @pl.loop(0, C, step=16)
        def _(i):
            ones_v[pl.ds(i, 16)] = jnp.ones((16,), _f32)

        @pl.loop(0, STRIPE, step=16)
        def _(i):
            zer_v[pl.ds(i, 16)] = jnp.zeros((16,), _f32)

        pltpu.sync_copy(zer_v, hist.at[pl.ds(sid * STRIPE, STRIPE)])
        plsc.subcore_barrier()

        @pl.loop(wid, NCHUNK, step=NW)
        def _(ci):
            pltpu.sync_copy(dst_hbm.at[pl.ds(ci * C, C)], idx_v)
            pltpu.sync_copy(ones_v, hist.at[idx_v], add=True)

        plsc.subcore_barrier()
        pltpu.sync_copy(hist.at[pl.ds(sid * STRIPE, STRIPE)],
                        out_hbm.at[cid, pl.ds(sid * STRIPE, STRIPE)])

    return deg_kernel(dst)


def _sc_gather_scatter(xs, src, dst):
    mesh = plsc.VectorSubcoreMesh(core_axis_name="c", subcore_axis_name="s")

    @functools.partial(
        pl.kernel,
        out_type=jax.ShapeDtypeStruct((NC, NPAD, D), _f32),
        mesh=mesh,
        scratch_types=[
            pltpu.VMEM_SHARED((NPAD, D), _f32),
            pltpu.VMEM((C,), jnp.int32),
            pltpu.VMEM((C,), jnp.int32),
            pltpu.VMEM((C, D), _f32),
        ],
    )
    def gs_kernel(xs_hbm, src_hbm, dst_hbm, out_hbm, acc, sidx, didx, rows):
        cid = lax.axis_index("c")
        sid = lax.axis_index("s")
        wid = cid * NS + sid

        @pl.loop(0, C)
        def _(r):
            @pl.loop(0, D, step=16)
            def _(c):
                rows[r, pl.ds(c, 16)] = jnp.zeros((16,), _f32)

        @pl.loop(0, STRIPE, step=C)
        def _(r):
            pltpu.sync_copy(rows, acc.at[pl.ds(sid * STRIPE + r, C)])

        plsc.subcore_barrier()

        @pl.loop(wid, NCHUNK, step=NW)
        def _(ci):
            pltpu.sync_copy(src_hbm.at[pl.ds(ci * C, C)], sidx)
            pltpu.sync_copy(dst_hbm.at[pl.ds(ci * C, C)], didx)
            pltpu.sync_copy(xs_hbm.at[sidx], rows)
            pltpu.sync_copy(rows, acc.at[didx], add=True)

        plsc.subcore_barrier()
        pltpu.sync_copy(acc.at[pl.ds(sid * STRIPE, STRIPE)],
                        out_hbm.at[cid, pl.ds(sid * STRIPE, STRIPE)])

    return gs_kernel(xs, src, dst)


def _tc_xw(x, w):
    def body(x_ref, w_ref, o_ref):
        o_ref[...] = jnp.dot(x_ref[...], w_ref[...],
                             preferred_element_type=_f32)

    return pl.pallas_call(
        body, out_shape=jax.ShapeDtypeStruct((N, D), _f32))(x, w)


def _tc_scale(xw, degp):
    def body(xw_ref, degp_ref, xs_ref, dinv_ref):
        deg = degp_ref[0, :N] + degp_ref[1, :N] + 1.0
        dinv = lax.rsqrt(deg)
        dinv_ref[...] = dinv[:, None]
        xs_ref[...] = xw_ref[...] * dinv[:, None]

    return pl.pallas_call(
        body,
        out_shape=(jax.ShapeDtypeStruct((N, D), _f32),
                   jax.ShapeDtypeStruct((N, 1), _f32)))(xw, degp)


def _tc_mid(x, xs, dinv, accp, b, g1, be1):
    def body(x_ref, xs_ref, dinv_ref, accp_ref, b_ref, g1_ref, be1_ref, h_ref):
        acc = accp_ref[0, :N, :] + accp_ref[1, :N, :] + xs_ref[...]
        h0 = dinv_ref[...] * acc + b_ref[...][None, :]
        m = jnp.mean(h0, axis=0)
        v = jnp.mean((h0 - m[None, :]) ** 2, axis=0)
        h = ((h0 - m[None, :]) * lax.rsqrt(v + 1e-5)[None, :]
             * g1_ref[...][None, :] + be1_ref[...][None, :])
        h_ref[...] = jnp.maximum(h, 0.0) + x_ref[...]

    return pl.pallas_call(
        body, out_shape=jax.ShapeDtypeStruct((N, D), _f32))(
            x, xs, dinv, accp, b, g1, be1)


def _tc_ffn(h, w1, b1, w2, b2, g2, be2):
    def body(h_ref, w1_ref, b1_ref, w2_ref, b2_ref, g2_ref, be2_ref, o_ref):
        hv = h_ref[...]
        f = jnp.dot(hv, w1_ref[...], preferred_element_type=_f32)
        f = jnp.maximum(f + b1_ref[...][None, :], 0.0)
        f = jnp.dot(f, w2_ref[...], preferred_element_type=_f32)
        t = f + b2_ref[...][None, :] + hv
        m = jnp.mean(t, axis=0)
        v = jnp.mean((t - m[None, :]) ** 2, axis=0)
        o_ref[...] = ((t - m[None, :]) * lax.rsqrt(v + 1e-5)[None, :]
                      * g2_ref[...][None, :] + be2_ref[...][None, :])

    return pl.pallas_call(
        body, out_shape=jax.ShapeDtypeStruct((N, D), _f32))(
            h, w1, b1, w2, b2, g2, be2)


@jax.jit
def kernel(x, edge_index, W_gcn, b_gcn, g1, be1, W1, b1, W2, b2, g2, be2):
    ei = edge_index.astype(jnp.int32)
    src = ei[0]
    dst = ei[1]
    degp = _sc_degree(dst)
    xw = _tc_xw(x, W_gcn)
    xs, dinv = _tc_scale(xw, degp)
    accp = _sc_gather_scatter(xs, src, dst)
    h = _tc_mid(x, xs, dinv, accp, b_gcn, g1, be1)
    return _tc_ffn(h, W1, b1, W2, b2, g2, be2)


# speedup vs baseline: 34.1463x; 1.6782x over previous
import functools

import jax
import jax.numpy as jnp
from jax import lax
from jax.experimental import pallas as pl
from jax.experimental.pallas import tpu as pltpu
from jax.experimental.pallas import tpu_sc as plsc

N = 10000
NPAD = 10240
D = 128
E = 320000
C = 128
NC = 2
NS = 16
NW = NC * NS
EPW = E // NW
NFULL = EPW // C
TAIL = EPW - NFULL * C
STRIPE = NPAD // NS

_f32 = jnp.float32


def _sc_degree(dst):
    mesh = plsc.VectorSubcoreMesh(core_axis_name="c", subcore_axis_name="s")

    @functools.partial(
        pl.kernel,
        out_type=jax.ShapeDtypeStruct((NC, NPAD), _f32),
        mesh=mesh,
        scratch_types=[
            pltpu.VMEM_SHARED((NPAD,), _f32),
            pltpu.VMEM((C,), jnp.int32),
            pltpu.VMEM((C,), jnp.int32),
            pltpu.VMEM((TAIL,), jnp.int32),
            pltpu.VMEM((C,), _f32),
            pltpu.VMEM((TAIL,), _f32),
            pltpu.VMEM((STRIPE,), _f32),
            pltpu.SemaphoreType.DMA,
            pltpu.SemaphoreType.DMA,
        ],
    )
    def deg_kernel(dst_hbm, out_hbm, hist, idx0, idx1, idxt, ones_v, onest_v,
                   zer_v, si0, si1):
        cid = lax.axis_index("c")
        sid = lax.axis_index("s")
        wid = cid * NS + sid
        base = wid * EPW
        idx = (idx0, idx1)
        sem = (si0, si1)

        @pl.loop(0, C, step=16)
        def _(i):
            ones_v[pl.ds(i, 16)] = jnp.ones((16,), _f32)

        @pl.loop(0, TAIL, step=16)
        def _(i):
            onest_v[pl.ds(i, 16)] = jnp.ones((16,), _f32)

        @pl.loop(0, STRIPE, step=16)
        def _(i):
            zer_v[pl.ds(i, 16)] = jnp.zeros((16,), _f32)

        pltpu.sync_copy(zer_v, hist.at[pl.ds(sid * STRIPE, STRIPE)])
        plsc.subcore_barrier()

        pltpu.sync_copy(dst_hbm.at[pl.ds(base, C)], idx0)
        pltpu.async_copy(dst_hbm.at[pl.ds(base + C, C)], idx1, si1)

        @pl.loop(0, NFULL // 2)
        def _(j):
            for b in (0, 1):
                k = 2 * j + b

                @pl.when(k > 0)
                def _():
                    pltpu.make_async_copy(
                        dst_hbm.at[pl.ds(base + k * C, C)], idx[b],
                        sem[b]).wait()

                pltpu.sync_copy(ones_v, hist.at[idx[b]], add=True)

                @pl.when(k + 2 < NFULL)
                def _():
                    pltpu.async_copy(
                        dst_hbm.at[pl.ds(base + (k + 2) * C, C)], idx[b],
                        sem[b])

        pltpu.sync_copy(dst_hbm.at[pl.ds(base + NFULL * C, TAIL)], idxt)
        pltpu.sync_copy(onest_v, hist.at[idxt], add=True)

        plsc.subcore_barrier()
        pltpu.sync_copy(hist.at[pl.ds(sid * STRIPE, STRIPE)],
                        out_hbm.at[cid, pl.ds(sid * STRIPE, STRIPE)])

    return deg_kernel(dst)


def _sc_gather_scatter(xs, src, dst):
    mesh = plsc.VectorSubcoreMesh(core_axis_name="c", subcore_axis_name="s")

    @functools.partial(
        pl.kernel,
        out_type=jax.ShapeDtypeStruct((NC, NPAD, D), _f32),
        mesh=mesh,
        scratch_types=[
            pltpu.VMEM_SHARED((NPAD, D), _f32),
            pltpu.VMEM((C,), jnp.int32),
            pltpu.VMEM((C,), jnp.int32),
            pltpu.VMEM((C,), jnp.int32),
            pltpu.VMEM((C,), jnp.int32),
            pltpu.VMEM((C, D), _f32),
            pltpu.VMEM((C, D), _f32),
            pltpu.VMEM((TAIL,), jnp.int32),
            pltpu.VMEM((TAIL,), jnp.int32),
            pltpu.VMEM((TAIL, D), _f32),
            pltpu.SemaphoreType.DMA,
            pltpu.SemaphoreType.DMA,
            pltpu.SemaphoreType.DMA,
            pltpu.SemaphoreType.DMA,
        ],
    )
    def gs_kernel(xs_hbm, src_hbm, dst_hbm, out_hbm, acc,
                  sidx0, sidx1, didx0, didx1, rows0, rows1,
                  sidxt, didxt, rowst, sg0, sg1, si0, si1):
        cid = lax.axis_index("c")
        sid = lax.axis_index("s")
        wid = cid * NS + sid
        base = wid * EPW
        sidx = (sidx0, sidx1)
        didx = (didx0, didx1)
        rows = (rows0, rows1)
        sg = (sg0, sg1)
        si = (si0, si1)

        @pl.loop(0, C)
        def _(r):
            @pl.loop(0, D, step=16)
            def _(c):
                rows0[r, pl.ds(c, 16)] = jnp.zeros((16,), _f32)

        @pl.loop(0, STRIPE, step=C)
        def _(r):
            pltpu.sync_copy(rows0, acc.at[pl.ds(sid * STRIPE + r, C)])

        plsc.subcore_barrier()

        pltpu.sync_copy(src_hbm.at[pl.ds(base, C)], sidx0)
        pltpu.sync_copy(dst_hbm.at[pl.ds(base, C)], didx0)
        pltpu.async_copy(xs_hbm.at[sidx0], rows0, sg0)
        pltpu.async_copy(src_hbm.at[pl.ds(base + C, C)], sidx1, si1)
        pltpu.async_copy(dst_hbm.at[pl.ds(base + C, C)], didx1, si1)

        @pl.loop(0, NFULL // 2)
        def _(j):
            for b in (0, 1):
                k = 2 * j + b
                nb = 1 - b

                @pl.when(k + 1 < NFULL)
                def _():
                    pltpu.make_async_copy(
                        src_hbm.at[pl.ds(base + (k + 1) * C, C)], sidx[nb],
                        si[nb]).wait()
                    pltpu.make_async_copy(
                        dst_hbm.at[pl.ds(base + (k + 1) * C, C)], didx[nb],
                        si[nb]).wait()
                    pltpu.async_copy(xs_hbm.at[sidx[nb]], rows[nb], sg[nb])

                pltpu.make_async_copy(xs_hbm.at[sidx[b]], rows[b],
                                      sg[b]).wait()
                pltpu.sync_copy(rows[b], acc.at[didx[b]], add=True)

                @pl.when(k + 2 < NFULL)
                def _():
                    pltpu.async_copy(
                        src_hbm.at[pl.ds(base + (k + 2) * C, C)], sidx[b],
                        si[b])
                    pltpu.async_copy(
                        dst_hbm.at[pl.ds(base + (k + 2) * C, C)], didx[b],
                        si[b])

        pltpu.sync_copy(src_hbm.at[pl.ds(base + NFULL * C, TAIL)], sidxt)
        pltpu.sync_copy(dst_hbm.at[pl.ds(base + NFULL * C, TAIL)], didxt)
        pltpu.sync_copy(xs_hbm.at[sidxt], rowst)
        pltpu.sync_copy(rowst, acc.at[didxt], add=True)

        plsc.subcore_barrier()
        pltpu.sync_copy(acc.at[pl.ds(sid * STRIPE, STRIPE)],
                        out_hbm.at[cid, pl.ds(sid * STRIPE, STRIPE)])

    return gs_kernel(xs, src, dst)


def _tc_xw(x, w):
    def body(x_ref, w_ref, o_ref):
        o_ref[...] = jnp.dot(x_ref[...], w_ref[...],
                             preferred_element_type=_f32)

    return pl.pallas_call(
        body, out_shape=jax.ShapeDtypeStruct((N, D), _f32))(x, w)


def _tc_scale(xw, degp):
    def body(xw_ref, degp_ref, xs_ref, dinv_ref):
        deg = degp_ref[0, :N] + degp_ref[1, :N] + 1.0
        dinv = lax.rsqrt(deg)
        dinv_ref[...] = dinv[:, None]
        xs_ref[...] = xw_ref[...] * dinv[:, None]

    return pl.pallas_call(
        body,
        out_shape=(jax.ShapeDtypeStruct((N, D), _f32),
                   jax.ShapeDtypeStruct((N, 1), _f32)))(xw, degp)


def _tc_mid(x, xs, dinv, accp, b, g1, be1):
    def body(x_ref, xs_ref, dinv_ref, accp_ref, b_ref, g1_ref, be1_ref, h_ref):
        acc = accp_ref[0, :N, :] + accp_ref[1, :N, :] + xs_ref[...]
        h0 = dinv_ref[...] * acc + b_ref[...][None, :]
        m = jnp.mean(h0, axis=0)
        v = jnp.mean((h0 - m[None, :]) ** 2, axis=0)
        h = ((h0 - m[None, :]) * lax.rsqrt(v + 1e-5)[None, :]
             * g1_ref[...][None, :] + be1_ref[...][None, :])
        h_ref[...] = jnp.maximum(h, 0.0) + x_ref[...]

    return pl.pallas_call(
        body, out_shape=jax.ShapeDtypeStruct((N, D), _f32))(
            x, xs, dinv, accp, b, g1, be1)


def _tc_ffn(h, w1, b1, w2, b2, g2, be2):
    def body(h_ref, w1_ref, b1_ref, w2_ref, b2_ref, g2_ref, be2_ref, o_ref):
        hv = h_ref[...]
        f = jnp.dot(hv, w1_ref[...], preferred_element_type=_f32)
        f = jnp.maximum(f + b1_ref[...][None, :], 0.0)
        f = jnp.dot(f, w2_ref[...], preferred_element_type=_f32)
        t = f + b2_ref[...][None, :] + hv
        m = jnp.mean(t, axis=0)
        v = jnp.mean((t - m[None, :]) ** 2, axis=0)
        o_ref[...] = ((t - m[None, :]) * lax.rsqrt(v + 1e-5)[None, :]
                      * g2_ref[...][None, :] + be2_ref[...][None, :])

    return pl.pallas_call(
        body, out_shape=jax.ShapeDtypeStruct((N, D), _f32))(
            h, w1, b1, w2, b2, g2, be2)


@jax.jit
def kernel(x, edge_index, W_gcn, b_gcn, g1, be1, W1, b1, W2, b2, g2, be2):
    ei = edge_index.astype(jnp.int32)
    src = ei[0]
    dst = ei[1]
    degp = _sc_degree(dst)
    xw = _tc_xw(x, W_gcn)
    xs, dinv = _tc_scale(xw, degp)
    accp = _sc_gather_scatter(xs, src, dst)
    h = _tc_mid(x, xs, dinv, accp, b_gcn, g1, be1)
    return _tc_ffn(h, W1, b1, W2, b2, g2, be2)
